# initial kernel scaffold (unmeasured)
import jax
import jax.numpy as jnp
from jax import lax
from jax.experimental import pallas as pl
from jax.experimental.pallas import tpu as pltpu

N_DEV = 16
STEPS = 2 * (N_DEV - 1)


def _gelu(y):
    c = 0.7978845608028654
    return 0.5 * y * (1.0 + jnp.tanh(c * (y + 0.044715 * y * y * y)))


def kernel(x, w_mat):
    m, k_per = x.shape
    k2, n = w_mat.shape
    assert k_per == k2
    chunk = m // N_DEV

    xb = x.astype(jnp.bfloat16)
    wb = w_mat.astype(jnp.bfloat16)

    def body(x_ref, w_ref, out_ref, send_buf, recv_buf, stage,
             send_sems, recv_sems, copy_sems, credit_sem):
        my = lax.axis_index("i")
        left = lax.rem(my + N_DEV - 1, N_DEV)
        right = lax.rem(my + 1, N_DEV)

        barrier = pltpu.get_barrier_semaphore()
        for nbr in (left, right):
            pl.semaphore_signal(barrier, inc=1, device_id=(nbr,),
                                device_id_type=pl.DeviceIdType.MESH)
        pl.semaphore_wait(barrier, 2)

        def partial_chunk(c):
            xa = x_ref[pl.ds(c * chunk, chunk), :]
            return jnp.dot(xa, w_ref[...], preferred_element_type=jnp.float32)

        send_buf[0, :, :] = partial_chunk(my).astype(jnp.bfloat16)

        pending = [None, None]

        def flush_out(slot2, val_f32, c):
            if pending[slot2] is not None:
                pending[slot2].wait()
            stage[slot2, :, :] = val_f32
            cp = pltpu.make_async_copy(
                stage.at[slot2],
                out_ref.at[pl.ds(c * chunk, chunk), :],
                copy_sems.at[slot2],
            )
            cp.start()
            pending[slot2] = cp

        for s in range(STEPS):
            slot = s % 2
            if s >= 2:
                pl.semaphore_wait(credit_sem, 1)
            rdma = pltpu.make_async_remote_copy(
                src_ref=send_buf.at[slot],
                dst_ref=recv_buf.at[slot],
                send_sem=send_sems.at[slot],
                recv_sem=recv_sems.at[slot],
                device_id=(right,),
                device_id_type=pl.DeviceIdType.MESH,
            )
            rdma.start()
            rdma.wait()

            if s < N_DEV - 1:
                c = lax.rem(my + 2 * N_DEV - s - 1, N_DEV)
                acc = recv_buf[slot].astype(jnp.float32) + partial_chunk(c)
                if s < N_DEV - 2:
                    send_buf[(s + 1) % 2, :, :] = acc.astype(jnp.bfloat16)
                else:
                    g = _gelu(acc)
                    send_buf[(s + 1) % 2, :, :] = g.astype(jnp.bfloat16)
                    flush_out(slot, g, c)
            else:
                t = s - (N_DEV - 1)
                c = lax.rem(my + 2 * N_DEV - t, N_DEV)
                rb = recv_buf[slot]
                if s < STEPS - 1:
                    send_buf[(s + 1) % 2, :, :] = rb
                flush_out(slot, rb.astype(jnp.float32), c)

            if s <= STEPS - 3:
                pl.semaphore_signal(credit_sem, inc=1, device_id=(left,),
                                    device_id_type=pl.DeviceIdType.MESH)

        for p in pending:
            if p is not None:
                p.wait()

    return pl.pallas_call(
        body,
        out_shape=jax.ShapeDtypeStruct((m, n), jnp.float32),
        in_specs=[
            pl.BlockSpec(memory_space=pltpu.VMEM),
            pl.BlockSpec(memory_space=pltpu.VMEM),
        ],
        out_specs=pl.BlockSpec(memory_space=pltpu.ANY),
        scratch_shapes=[
            pltpu.VMEM((2, chunk, n), jnp.bfloat16),
            pltpu.VMEM((2, chunk, n), jnp.bfloat16),
            pltpu.VMEM((2, chunk, n), jnp.float32),
            pltpu.SemaphoreType.DMA((2,)),
            pltpu.SemaphoreType.DMA((2,)),
            pltpu.SemaphoreType.DMA((2,)),
            pltpu.SemaphoreType.REGULAR,
        ],
        compiler_params=pltpu.CompilerParams(
            collective_id=0,
            vmem_limit_bytes=128 * 1024 * 1024,
        ),
    )(xb, wb)


# baseline (device time: 1551513 ns/iter reference)
import jax
import jax.numpy as jnp
from jax import lax
from jax.experimental import pallas as pl
from jax.experimental.pallas import tpu as pltpu

N_DEV = 16
STEPS = 2 * (N_DEV - 1)


def _gelu(y):
    c = 0.7978845608028654
    return 0.5 * y * (1.0 + jnp.tanh(c * (y + 0.044715 * y * y * y)))


def kernel(x, w_mat):
    m, k_per = x.shape
    k2, n = w_mat.shape
    assert k_per == k2
    chunk = m // N_DEV

    xb = x.astype(jnp.bfloat16)
    wb = w_mat.astype(jnp.bfloat16)

    def body(x_ref, w_ref, out_ref, send_buf, recv_buf, stage,
             send_sems, recv_sems, copy_sems, credit_sem):
        my = lax.axis_index("i")
        left = lax.rem(my + N_DEV - 1, N_DEV)
        right = lax.rem(my + 1, N_DEV)

        barrier = pltpu.get_barrier_semaphore()
        for nbr in (left, right):
            pl.semaphore_signal(barrier, inc=1, device_id=(nbr,),
                                device_id_type=pl.DeviceIdType.MESH)
        pl.semaphore_wait(barrier, 2)

        def partial_chunk(c):
            xa = x_ref[pl.ds(c * chunk, chunk), :]
            return jnp.dot(xa, w_ref[...], preferred_element_type=jnp.float32)

        send_buf[0, :, :] = partial_chunk(my).astype(jnp.bfloat16)

        pending = [None, None]

        def flush_out(slot2, val_f32, c):
            if pending[slot2] is not None:
                pending[slot2].wait()
            stage[slot2, :, :] = val_f32
            cp = pltpu.make_async_copy(
                stage.at[slot2],
                out_ref.at[pl.ds(c * chunk, chunk), :],
                copy_sems.at[slot2],
            )
            cp.start()
            pending[slot2] = cp

        for s in range(STEPS):
            slot = s % 2
            if s >= 2:
                pl.semaphore_wait(credit_sem, 1)
            rdma = pltpu.make_async_remote_copy(
                src_ref=send_buf.at[slot],
                dst_ref=recv_buf.at[slot],
                send_sem=send_sems.at[slot],
                recv_sem=recv_sems.at[slot],
                device_id=(right,),
                device_id_type=pl.DeviceIdType.MESH,
            )
            rdma.start()
            rdma.wait()

            if s < N_DEV - 1:
                c = lax.rem(my + 2 * N_DEV - s - 1, N_DEV)
                acc = recv_buf[slot].astype(jnp.float32) + partial_chunk(c)
                if s < N_DEV - 2:
                    send_buf[(s + 1) % 2, :, :] = acc.astype(jnp.bfloat16)
                else:
                    g = _gelu(acc)
                    send_buf[(s + 1) % 2, :, :] = g.astype(jnp.bfloat16)
                    flush_out(slot, g, c)
            else:
                t = s - (N_DEV - 1)
                c = lax.rem(my + 2 * N_DEV - t, N_DEV)
                rb = recv_buf[slot]
                if s < STEPS - 1:
                    send_buf[(s + 1) % 2, :, :] = rb
                flush_out(slot, rb.astype(jnp.float32), c)

            if s <= STEPS - 3:
                pl.semaphore_signal(credit_sem, inc=1, device_id=(left,),
                                    device_id_type=pl.DeviceIdType.MESH)

        for p in pending:
            if p is not None:
                p.wait()

    return pl.pallas_call(
        body,
        out_shape=jax.ShapeDtypeStruct((m, n), jnp.float32),
        in_specs=[
            pl.BlockSpec(memory_space=pltpu.VMEM),
            pl.BlockSpec(memory_space=pltpu.VMEM),
        ],
        out_specs=pl.BlockSpec(memory_space=pl.ANY),
        scratch_shapes=[
            pltpu.VMEM((2, chunk, n), jnp.bfloat16),
            pltpu.VMEM((2, chunk, n), jnp.bfloat16),
            pltpu.VMEM((2, chunk, n), jnp.float32),
            pltpu.SemaphoreType.DMA((2,)),
            pltpu.SemaphoreType.DMA((2,)),
            pltpu.SemaphoreType.DMA((2,)),
            pltpu.SemaphoreType.REGULAR,
        ],
        compiler_params=pltpu.CompilerParams(
            collective_id=0,
            vmem_limit_bytes=128 * 1024 * 1024,
        ),
    )(xb, wb)


# device time: 905556 ns/iter; 1.7133x vs baseline; 1.7133x over previous
import jax
import jax.numpy as jnp
from jax import lax
from jax.experimental import pallas as pl
from jax.experimental.pallas import tpu as pltpu

N_DEV = 16
STEPS = 2 * (N_DEV - 1)


def _gelu(y):
    c = 0.7978845608028654
    return 0.5 * y * (1.0 + jnp.tanh(c * (y + 0.044715 * y * y * y)))


def kernel(x, w_mat):
    m, k_per = x.shape
    k2, n = w_mat.shape
    assert k_per == k2
    chunk = m // N_DEV
    nh = n // 2

    xb = x.astype(jnp.bfloat16)
    wb = w_mat.astype(jnp.bfloat16)

    def body(x_ref, w_ref, out_ref,
             send_a, recv_a, send_b, recv_b, stage_a, stage_b,
             send_sems_a, recv_sems_a, send_sems_b, recv_sems_b,
             copy_sems_a, copy_sems_b, credit_a, credit_b):
        my = lax.axis_index("i")
        left = lax.rem(my + N_DEV - 1, N_DEV)
        right = lax.rem(my + 1, N_DEV)

        barrier = pltpu.get_barrier_semaphore()
        for nbr in (left, right):
            pl.semaphore_signal(barrier, inc=1, device_id=(nbr,),
                                device_id_type=pl.DeviceIdType.MESH)
        pl.semaphore_wait(barrier, 2)

        def partial_chunk(c, col0):
            xa = x_ref[pl.ds(c * chunk, chunk), :]
            wa = w_ref[:, pl.ds(col0, nh)]
            return jnp.dot(xa, wa, preferred_element_type=jnp.float32)

        dirs = (
            dict(send_buf=send_a, recv_buf=recv_a, stage=stage_a,
                 send_sems=send_sems_a, recv_sems=recv_sems_a,
                 copy_sems=copy_sems_a, credit=credit_a,
                 dst=right, credit_dst=left, col0=0, sign=-1,
                 pending=[None, None]),
            dict(send_buf=send_b, recv_buf=recv_b, stage=stage_b,
                 send_sems=send_sems_b, recv_sems=recv_sems_b,
                 copy_sems=copy_sems_b, credit=credit_b,
                 dst=left, credit_dst=right, col0=nh, sign=1,
                 pending=[None, None]),
        )

        for D in dirs:
            D["send_buf"][0, :, :] = (
                partial_chunk(my, D["col0"]).astype(jnp.bfloat16))

        def flush_out(D, slot, val_f32, c):
            if D["pending"][slot] is not None:
                D["pending"][slot].wait()
            D["stage"][slot, :, :] = val_f32
            cp = pltpu.make_async_copy(
                D["stage"].at[slot],
                out_ref.at[pl.ds(c * chunk, chunk), pl.ds(D["col0"], nh)],
                D["copy_sems"].at[slot],
            )
            cp.start()
            D["pending"][slot] = cp

        for s in range(STEPS):
            slot = s % 2
            rdmas = []
            for D in dirs:
                if s >= 2:
                    pl.semaphore_wait(D["credit"], 1)
                rdma = pltpu.make_async_remote_copy(
                    src_ref=D["send_buf"].at[slot],
                    dst_ref=D["recv_buf"].at[slot],
                    send_sem=D["send_sems"].at[slot],
                    recv_sem=D["recv_sems"].at[slot],
                    device_id=(D["dst"],),
                    device_id_type=pl.DeviceIdType.MESH,
                )
                rdma.start()
                rdmas.append(rdma)

            if s < N_DEV - 1:
                for D, rdma in zip(dirs, rdmas):
                    c = lax.rem(my + 2 * N_DEV + D["sign"] * (s + 1), N_DEV)
                    p = partial_chunk(c, D["col0"])
                    rdma.wait()
                    acc = D["recv_buf"][slot].astype(jnp.float32) + p
                    if s < N_DEV - 2:
                        D["send_buf"][(s + 1) % 2, :, :] = (
                            acc.astype(jnp.bfloat16))
                    else:
                        g = _gelu(acc)
                        D["send_buf"][(s + 1) % 2, :, :] = (
                            g.astype(jnp.bfloat16))
                        flush_out(D, slot, g, c)
            else:
                t = s - (N_DEV - 1)
                for D, rdma in zip(dirs, rdmas):
                    c = lax.rem(my + 2 * N_DEV + D["sign"] * t, N_DEV)
                    rdma.wait()
                    rb = D["recv_buf"][slot]
                    if s < STEPS - 1:
                        D["send_buf"][(s + 1) % 2, :, :] = rb
                    flush_out(D, slot, rb.astype(jnp.float32), c)

            if s <= STEPS - 3:
                for D in dirs:
                    pl.semaphore_signal(D["credit"], inc=1,
                                        device_id=(D["credit_dst"],),
                                        device_id_type=pl.DeviceIdType.MESH)

        for D in dirs:
            for p in D["pending"]:
                if p is not None:
                    p.wait()

    half = (2, chunk, nh)
    return pl.pallas_call(
        body,
        out_shape=jax.ShapeDtypeStruct((m, n), jnp.float32),
        in_specs=[
            pl.BlockSpec(memory_space=pltpu.VMEM),
            pl.BlockSpec(memory_space=pltpu.VMEM),
        ],
        out_specs=pl.BlockSpec(memory_space=pl.ANY),
        scratch_shapes=[
            pltpu.VMEM(half, jnp.bfloat16),
            pltpu.VMEM(half, jnp.bfloat16),
            pltpu.VMEM(half, jnp.bfloat16),
            pltpu.VMEM(half, jnp.bfloat16),
            pltpu.VMEM(half, jnp.float32),
            pltpu.VMEM(half, jnp.float32),
            pltpu.SemaphoreType.DMA((2,)),
            pltpu.SemaphoreType.DMA((2,)),
            pltpu.SemaphoreType.DMA((2,)),
            pltpu.SemaphoreType.DMA((2,)),
            pltpu.SemaphoreType.DMA((2,)),
            pltpu.SemaphoreType.DMA((2,)),
            pltpu.SemaphoreType.REGULAR,
            pltpu.SemaphoreType.REGULAR,
        ],
        compiler_params=pltpu.CompilerParams(
            collective_id=0,
            vmem_limit_bytes=128 * 1024 * 1024,
        ),
    )(xb, wb)


# device time: 896899 ns/iter; 1.7299x vs baseline; 1.0097x over previous
import jax
import jax.numpy as jnp
from jax import lax
from jax.experimental import pallas as pl
from jax.experimental.pallas import tpu as pltpu

N_DEV = 16
STEPS = 2 * (N_DEV - 1)


def _gelu(y):
    c = 0.7978845608028654
    return 0.5 * y * (1.0 + jnp.tanh(c * (y + 0.044715 * y * y * y)))


def kernel(x, w_mat):
    m, k_per = x.shape
    k2, n = w_mat.shape
    assert k_per == k2
    chunk = m // N_DEV
    nh = n // 2

    xb = x.astype(jnp.bfloat16)
    wb = w_mat.astype(jnp.bfloat16)

    def body(x_ref, w_ref, out_ref,
             send_a, recv_a, send_b, recv_b, stage_a, stage_b,
             send_sems_a, recv_sems_a, send_sems_b, recv_sems_b,
             copy_sems_a, copy_sems_b, credit_a, credit_b):
        my = lax.axis_index("i")
        left = lax.rem(my + N_DEV - 1, N_DEV)
        right = lax.rem(my + 1, N_DEV)

        barrier = pltpu.get_barrier_semaphore()
        for nbr in (left, right):
            pl.semaphore_signal(barrier, inc=1, device_id=(nbr,),
                                device_id_type=pl.DeviceIdType.MESH)
        pl.semaphore_wait(barrier, 2)

        def partial_chunk(c, col0):
            xa = x_ref[pl.ds(c * chunk, chunk), :]
            wa = w_ref[:, pl.ds(col0, nh)]
            return jnp.dot(xa, wa, preferred_element_type=jnp.float32)

        dirs = (
            dict(send_buf=send_a, recv_buf=recv_a, stage=stage_a,
                 send_sems=send_sems_a, recv_sems=recv_sems_a,
                 copy_sems=copy_sems_a, credit=credit_a,
                 dst=right, credit_dst=left, col0=0, sign=-1,
                 pending=[None, None]),
            dict(send_buf=send_b, recv_buf=recv_b, stage=stage_b,
                 send_sems=send_sems_b, recv_sems=recv_sems_b,
                 copy_sems=copy_sems_b, credit=credit_b,
                 dst=left, credit_dst=right, col0=nh, sign=1,
                 pending=[None, None]),
        )

        for D in dirs:
            D["send_buf"][0, :, :] = (
                partial_chunk(my, D["col0"]).astype(jnp.bfloat16))

        def flush_out(D, slot, val_f32, c):
            if D["pending"][slot] is not None:
                D["pending"][slot].wait()
            D["stage"][slot, :, :] = val_f32
            cp = pltpu.make_async_copy(
                D["stage"].at[slot],
                out_ref.at[pl.ds(c * chunk, chunk), pl.ds(D["col0"], nh)],
                D["copy_sems"].at[slot],
            )
            cp.start()
            D["pending"][slot] = cp

        def run_deferred(batch):
            for D, dslot, c, ds in batch:
                flush_out(D, dslot,
                          D["recv_buf"][dslot].astype(jnp.float32), c)
                if ds <= STEPS - 3:
                    pl.semaphore_signal(D["credit"], inc=1,
                                        device_id=(D["credit_dst"],),
                                        device_id_type=pl.DeviceIdType.MESH)

        deferred = []
        for s in range(STEPS):
            slot = s % 2
            rdmas = []
            for D in dirs:
                if s >= 2:
                    pl.semaphore_wait(D["credit"], 1)
                rdma = pltpu.make_async_remote_copy(
                    src_ref=D["send_buf"].at[slot],
                    dst_ref=D["recv_buf"].at[slot],
                    send_sem=D["send_sems"].at[slot],
                    recv_sem=D["recv_sems"].at[slot],
                    device_id=(D["dst"],),
                    device_id_type=pl.DeviceIdType.MESH,
                )
                rdma.start()
                rdmas.append(rdma)

            run_deferred(deferred)
            deferred = []

            if s < N_DEV - 1:
                for D, rdma in zip(dirs, rdmas):
                    c = lax.rem(my + 2 * N_DEV + D["sign"] * (s + 1), N_DEV)
                    p = partial_chunk(c, D["col0"])
                    rdma.wait()
                    acc = D["recv_buf"][slot].astype(jnp.float32) + p
                    if s < N_DEV - 2:
                        D["send_buf"][(s + 1) % 2, :, :] = (
                            acc.astype(jnp.bfloat16))
                    else:
                        g = _gelu(acc)
                        D["send_buf"][(s + 1) % 2, :, :] = (
                            g.astype(jnp.bfloat16))
                        flush_out(D, slot, g, c)
            else:
                t = s - (N_DEV - 1)
                for D, rdma in zip(dirs, rdmas):
                    c = lax.rem(my + 2 * N_DEV + D["sign"] * t, N_DEV)
                    rdma.wait()
                    if s < STEPS - 1:
                        D["send_buf"][(s + 1) % 2, :, :] = (
                            D["recv_buf"][slot])
                    deferred.append((D, slot, c, s))

            if s < N_DEV - 1:
                for D in dirs:
                    pl.semaphore_signal(D["credit"], inc=1,
                                        device_id=(D["credit_dst"],),
                                        device_id_type=pl.DeviceIdType.MESH)

        run_deferred(deferred)

        for D in dirs:
            for p in D["pending"]:
                if p is not None:
                    p.wait()

    half = (2, chunk, nh)
    return pl.pallas_call(
        body,
        out_shape=jax.ShapeDtypeStruct((m, n), jnp.float32),
        in_specs=[
            pl.BlockSpec(memory_space=pltpu.VMEM),
            pl.BlockSpec(memory_space=pltpu.VMEM),
        ],
        out_specs=pl.BlockSpec(memory_space=pl.ANY),
        scratch_shapes=[
            pltpu.VMEM(half, jnp.bfloat16),
            pltpu.VMEM(half, jnp.bfloat16),
            pltpu.VMEM(half, jnp.bfloat16),
            pltpu.VMEM(half, jnp.bfloat16),
            pltpu.VMEM(half, jnp.float32),
            pltpu.VMEM(half, jnp.float32),
            pltpu.SemaphoreType.DMA((2,)),
            pltpu.SemaphoreType.DMA((2,)),
            pltpu.SemaphoreType.DMA((2,)),
            pltpu.SemaphoreType.DMA((2,)),
            pltpu.SemaphoreType.DMA((2,)),
            pltpu.SemaphoreType.DMA((2,)),
            pltpu.SemaphoreType.REGULAR,
            pltpu.SemaphoreType.REGULAR,
        ],
        compiler_params=pltpu.CompilerParams(
            collective_id=0,
            vmem_limit_bytes=128 * 1024 * 1024,
        ),
    )(xb, wb)


# device time: 784287 ns/iter; 1.9782x vs baseline; 1.1436x over previous
import jax
import jax.numpy as jnp
from jax import lax
from jax.experimental import pallas as pl
from jax.experimental.pallas import tpu as pltpu

N_DEV = 16
STEPS = 2 * (N_DEV - 1)
N_LANES = 4


def _gelu(y):
    c = 0.7978845608028654
    return 0.5 * y * (1.0 + jnp.tanh(c * (y + 0.044715 * y * y * y)))


def kernel(x, w_mat):
    m, k_per = x.shape
    k2, n = w_mat.shape
    assert k_per == k2
    chunk = m // N_DEV
    nq = n // N_LANES

    xb = x.astype(jnp.bfloat16)
    wb = w_mat.astype(jnp.bfloat16)

    def body(x_ref, w_ref, out_ref, *scratch):
        send_bufs = scratch[0:4]
        recv_bufs = scratch[4:8]
        stages = scratch[8:12]
        send_sems = scratch[12:16]
        recv_sems = scratch[16:20]
        copy_sems = scratch[20:24]
        credits = scratch[24:28]

        my = lax.axis_index("i")
        left = lax.rem(my + N_DEV - 1, N_DEV)
        right = lax.rem(my + 1, N_DEV)

        barrier = pltpu.get_barrier_semaphore()
        for nbr in (left, right):
            pl.semaphore_signal(barrier, inc=1, device_id=(nbr,),
                                device_id_type=pl.DeviceIdType.MESH)
        pl.semaphore_wait(barrier, 2)

        def partial_chunk(c, col0):
            xa = x_ref[pl.ds(c * chunk, chunk), :]
            wa = w_ref[:, pl.ds(col0, nq)]
            return jnp.dot(xa, wa, preferred_element_type=jnp.float32)

        lanes = []
        for li, (quarter, dst, credit_dst, sign) in enumerate((
                (0, right, left, -1),
                (2, left, right, 1),
                (1, right, left, -1),
                (3, left, right, 1),
        )):
            lanes.append(dict(
                send_buf=send_bufs[quarter], recv_buf=recv_bufs[quarter],
                stage=stages[quarter], send_sems=send_sems[quarter],
                recv_sems=recv_sems[quarter], copy_sems=copy_sems[quarter],
                credit=credits[quarter], dst=dst, credit_dst=credit_dst,
                col0=quarter * nq, sign=sign,
                pending=[None, None], rdma=None,
            ))

        def start_hop(L, s):
            if s >= 2:
                pl.semaphore_wait(L["credit"], 1)
            slot = s % 2
            rdma = pltpu.make_async_remote_copy(
                src_ref=L["send_buf"].at[slot],
                dst_ref=L["recv_buf"].at[slot],
                send_sem=L["send_sems"].at[slot],
                recv_sem=L["recv_sems"].at[slot],
                device_id=(L["dst"],),
                device_id_type=pl.DeviceIdType.MESH,
            )
            rdma.start()
            L["rdma"] = rdma

        def flush_out(L, slot, val_f32, c):
            if L["pending"][slot] is not None:
                L["pending"][slot].wait()
            L["stage"][slot, :, :] = val_f32
            cp = pltpu.make_async_copy(
                L["stage"].at[slot],
                out_ref.at[pl.ds(c * chunk, chunk), pl.ds(L["col0"], nq)],
                L["copy_sems"].at[slot],
            )
            cp.start()
            L["pending"][slot] = cp

        def send_credit(L):
            pl.semaphore_signal(L["credit"], inc=1,
                                device_id=(L["credit_dst"],),
                                device_id_type=pl.DeviceIdType.MESH)

        for L in lanes:
            L["send_buf"][0, :, :] = (
                partial_chunk(my, L["col0"]).astype(jnp.bfloat16))
            start_hop(L, 0)

        def lane_step(L, s):
            slot = s % 2
            if s < N_DEV - 1:
                c = lax.rem(my + 2 * N_DEV + L["sign"] * (s + 1), N_DEV)
                p = partial_chunk(c, L["col0"])
                L["rdma"].wait()
                acc = L["recv_buf"][slot].astype(jnp.float32) + p
                if s < N_DEV - 2:
                    L["send_buf"][(s + 1) % 2, :, :] = (
                        acc.astype(jnp.bfloat16))
                    start_hop(L, s + 1)
                    send_credit(L)
                else:
                    g = _gelu(acc)
                    L["send_buf"][(s + 1) % 2, :, :] = (
                        g.astype(jnp.bfloat16))
                    start_hop(L, s + 1)
                    flush_out(L, slot, g, c)
                    send_credit(L)
            else:
                t = s - (N_DEV - 1)
                c = lax.rem(my + 2 * N_DEV + L["sign"] * t, N_DEV)
                L["rdma"].wait()
                if s < STEPS - 1:
                    L["send_buf"][(s + 1) % 2, :, :] = L["recv_buf"][slot]
                    start_hop(L, s + 1)
                flush_out(L, slot, L["recv_buf"][slot].astype(jnp.float32),
                          c)
                if s <= STEPS - 3:
                    send_credit(L)

        for s in range(STEPS):
            for L in lanes:
                lane_step(L, s)

        for L in lanes:
            for p in L["pending"]:
                if p is not None:
                    p.wait()

    quarter_buf = (2, chunk, nq)
    scratch_shapes = (
        [pltpu.VMEM(quarter_buf, jnp.bfloat16)] * 4 +
        [pltpu.VMEM(quarter_buf, jnp.bfloat16)] * 4 +
        [pltpu.VMEM(quarter_buf, jnp.float32)] * 4 +
        [pltpu.SemaphoreType.DMA((2,))] * 4 +
        [pltpu.SemaphoreType.DMA((2,))] * 4 +
        [pltpu.SemaphoreType.DMA((2,))] * 4 +
        [pltpu.SemaphoreType.REGULAR] * 4
    )
    return pl.pallas_call(
        body,
        out_shape=jax.ShapeDtypeStruct((m, n), jnp.float32),
        in_specs=[
            pl.BlockSpec(memory_space=pltpu.VMEM),
            pl.BlockSpec(memory_space=pltpu.VMEM),
        ],
        out_specs=pl.BlockSpec(memory_space=pl.ANY),
        scratch_shapes=scratch_shapes,
        compiler_params=pltpu.CompilerParams(
            collective_id=0,
            vmem_limit_bytes=128 * 1024 * 1024,
        ),
    )(xb, wb)


# device time: 784265 ns/iter; 1.9783x vs baseline; 1.0000x over previous
import jax
import jax.numpy as jnp
from jax import lax
from jax.experimental import pallas as pl
from jax.experimental.pallas import tpu as pltpu

N_DEV = 16
STEPS = 2 * (N_DEV - 1)
N_LANES = 4
SLOTS = 3


def _gelu(y):
    c = 0.7978845608028654
    return 0.5 * y * (1.0 + jnp.tanh(c * (y + 0.044715 * y * y * y)))


def kernel(x, w_mat):
    m, k_per = x.shape
    k2, n = w_mat.shape
    assert k_per == k2
    chunk = m // N_DEV
    nq = n // N_LANES

    xb = x.astype(jnp.bfloat16)
    wb = w_mat.astype(jnp.bfloat16)

    def body(x_ref, w_ref, out_ref, *scratch):
        send_bufs = scratch[0:4]
        recv_bufs = scratch[4:8]
        stages = scratch[8:12]
        send_sems = scratch[12:16]
        recv_sems = scratch[16:20]
        copy_sems = scratch[20:24]
        credits = scratch[24:28]

        my = lax.axis_index("i")
        left = lax.rem(my + N_DEV - 1, N_DEV)
        right = lax.rem(my + 1, N_DEV)

        barrier = pltpu.get_barrier_semaphore()
        for nbr in (left, right):
            pl.semaphore_signal(barrier, inc=1, device_id=(nbr,),
                                device_id_type=pl.DeviceIdType.MESH)
        pl.semaphore_wait(barrier, 2)

        def partial_chunk(c, col0):
            xa = x_ref[pl.ds(c * chunk, chunk), :]
            wa = w_ref[:, pl.ds(col0, nq)]
            return jnp.dot(xa, wa,
                           preferred_element_type=jnp.float32
                           ).astype(jnp.bfloat16)

        lanes = []
        for quarter, dst, credit_dst, sign in (
                (0, right, left, -1),
                (2, left, right, 1),
                (1, right, left, -1),
                (3, left, right, 1),
        ):
            lanes.append(dict(
                send_buf=send_bufs[quarter], recv_buf=recv_bufs[quarter],
                stage=stages[quarter], send_sems=send_sems[quarter],
                recv_sems=recv_sems[quarter], copy_sems=copy_sems[quarter],
                credit=credits[quarter], dst=dst, credit_dst=credit_dst,
                col0=quarter * nq, sign=sign,
                pending=[None, None], nflush=0, rdma=None,
            ))

        def start_hop(L, s, src):
            if s >= SLOTS:
                pl.semaphore_wait(L["credit"], 1)
            slot = s % SLOTS
            rdma = pltpu.make_async_remote_copy(
                src_ref=src,
                dst_ref=L["recv_buf"].at[slot],
                send_sem=L["send_sems"].at[slot],
                recv_sem=L["recv_sems"].at[slot],
                device_id=(L["dst"],),
                device_id_type=pl.DeviceIdType.MESH,
            )
            rdma.start()
            L["rdma"] = rdma

        def flush_out(L, val_f32, c):
            fs = L["nflush"] % 2
            L["nflush"] += 1
            if L["pending"][fs] is not None:
                L["pending"][fs].wait()
            L["stage"][fs, :, :] = val_f32
            cp = pltpu.make_async_copy(
                L["stage"].at[fs],
                out_ref.at[pl.ds(c * chunk, chunk), pl.ds(L["col0"], nq)],
                L["copy_sems"].at[fs],
            )
            cp.start()
            L["pending"][fs] = cp

        def send_credit(L):
            pl.semaphore_signal(L["credit"], inc=1,
                                device_id=(L["credit_dst"],),
                                device_id_type=pl.DeviceIdType.MESH)

        for L in lanes:
            L["send_buf"][0, :, :] = partial_chunk(my, L["col0"])
            start_hop(L, 0, L["send_buf"].at[0])

        def lane_step(L, s):
            slot = s % SLOTS
            if s < N_DEV - 1:
                c = lax.rem(my + 2 * N_DEV + L["sign"] * (s + 1), N_DEV)
                p = partial_chunk(c, L["col0"])
                L["rdma"].wait()
                if s < N_DEV - 2:
                    L["send_buf"][(s + 1) % SLOTS, :, :] = (
                        L["recv_buf"][slot] + p)
                    start_hop(L, s + 1, L["send_buf"].at[(s + 1) % SLOTS])
                    send_credit(L)
                else:
                    g = _gelu((L["recv_buf"][slot] + p)
                              .astype(jnp.float32))
                    L["send_buf"][(s + 1) % SLOTS, :, :] = (
                        g.astype(jnp.bfloat16))
                    start_hop(L, s + 1, L["send_buf"].at[(s + 1) % SLOTS])
                    send_credit(L)
                    flush_out(L, g, c)
            else:
                t = s - (N_DEV - 1)
                c = lax.rem(my + 2 * N_DEV + L["sign"] * t, N_DEV)
                L["rdma"].wait()
                if s < STEPS - 1:
                    start_hop(L, s + 1, L["recv_buf"].at[slot])
                if N_DEV <= s <= STEPS - 3:
                    send_credit(L)
                flush_out(L, L["recv_buf"][slot].astype(jnp.float32), c)

        for s in range(STEPS):
            for L in lanes:
                lane_step(L, s)

        for L in lanes:
            for p in L["pending"]:
                if p is not None:
                    p.wait()

    comm_buf = (SLOTS, chunk, nq)
    stage_buf = (2, chunk, nq)
    scratch_shapes = (
        [pltpu.VMEM(comm_buf, jnp.bfloat16)] * 4 +
        [pltpu.VMEM(comm_buf, jnp.bfloat16)] * 4 +
        [pltpu.VMEM(stage_buf, jnp.float32)] * 4 +
        [pltpu.SemaphoreType.DMA((SLOTS,))] * 4 +
        [pltpu.SemaphoreType.DMA((SLOTS,))] * 4 +
        [pltpu.SemaphoreType.DMA((2,))] * 4 +
        [pltpu.SemaphoreType.REGULAR] * 4
    )
    return pl.pallas_call(
        body,
        out_shape=jax.ShapeDtypeStruct((m, n), jnp.float32),
        in_specs=[
            pl.BlockSpec(memory_space=pltpu.VMEM),
            pl.BlockSpec(memory_space=pltpu.VMEM),
        ],
        out_specs=pl.BlockSpec(memory_space=pl.ANY),
        scratch_shapes=scratch_shapes,
        compiler_params=pltpu.CompilerParams(
            collective_id=0,
            vmem_limit_bytes=128 * 1024 * 1024,
        ),
    )(xb, wb)
